# Initial kernel scaffold; baseline (speedup 1.0000x reference)
import jax
import jax.numpy as jnp
from jax.experimental import pallas as pl


def kernel(x, W_e0, b_e0, W_e1, b_e1, W_mu, b_mu, W_lv, b_lv, W_d, b_d, W_d0, b_d0, W_d1, b_d1, W_d2, b_d2, emb, eps):
    raise NotImplementedError("write your pallas kernel here")


# speedup vs baseline: 5.2708x; 5.2708x over previous
import jax
import jax.numpy as jnp
from jax.experimental import pallas as pl

_N = 16384
_D = 256
_K = 256
_B = 1024


def _lrelu(v):
    return jnp.where(v >= 0, v, 0.01 * v)


def _body(x_ref, eps_ref, w0_ref, b0_ref, W1_ref, b1_ref, Wmu_ref, bmu_ref,
          Wlv_ref, blv_ref, embT_ref, emb_ref, e2_ref,
          Wd_ref, bd_ref, Wd0_ref, bd0_ref, Wd1_ref, bd1_ref, Wd2_ref, bd2_ref,
          ze_ref, zq_ref, de_ref, dq_ref, cs_ref, ss_ref):
    xb = x_ref[...]
    h = _lrelu(xb * w0_ref[...] + b0_ref[...])
    h = _lrelu(jnp.dot(h, W1_ref[...]) + b1_ref[...])
    mu = jnp.dot(h, Wmu_ref[...]) + bmu_ref[...]
    lv = jnp.dot(h, Wlv_ref[...]) + blv_ref[...]
    ze = mu + eps_ref[...] * jnp.exp(0.5 * lv)
    ze_ref[...] = ze

    dots = jnp.dot(ze, embT_ref[...])
    z2 = jnp.sum(ze * ze, axis=1, keepdims=True)
    d = (z2 - 2.0 * dots) + e2_ref[...]
    dmin = jnp.min(d, axis=1, keepdims=True)
    j = jax.lax.broadcasted_iota(jnp.int32, d.shape, 1)
    k = jnp.min(jnp.where(d == dmin, j, _K), axis=1, keepdims=True)

    cs_part = jnp.sum(dmin)

    k1, k2 = k // 16, k % 16
    n_up = jnp.clip(k1 - 1, 0, 15) * 16 + k2
    n_dn = jnp.clip(k1 + 1, 0, 15) * 16 + k2
    n_lf = k1 * 16 + jnp.clip(k2 - 1, 0, 15)
    n_rt = k1 * 16 + jnp.clip(k2 + 1, 0, 15)
    m = ((j == n_up).astype(jnp.float32) + (j == n_dn).astype(jnp.float32)
         + (j == n_lf).astype(jnp.float32) + (j == n_rt).astype(jnp.float32))
    ss_part = jnp.sum(m * d)

    oh = (j == k).astype(jnp.float32)
    zq = jnp.dot(oh, emb_ref[...])
    zq_ref[...] = zq

    def dec(z):
        y = _lrelu(jnp.dot(z, Wd_ref[...]) + bd_ref[...])
        y = _lrelu(jnp.dot(y, Wd0_ref[...]) + bd0_ref[...])
        y = _lrelu(jnp.dot(y, Wd1_ref[...]) + bd1_ref[...])
        y = _lrelu(jnp.dot(y, Wd2_ref[...]) + bd2_ref[...])
        return y

    de_ref[...] = dec(ze)[:, 0:1]
    dq_ref[...] = dec(zq)[:, 0:1]

    @pl.when(pl.program_id(0) == 0)
    def _init():
        cs_ref[...] = jnp.zeros_like(cs_ref)
        ss_ref[...] = jnp.zeros_like(ss_ref)

    cs_ref[...] += cs_part
    ss_ref[...] += ss_part


def kernel(x, W_e0, b_e0, W_e1, b_e1, W_mu, b_mu, W_lv, b_lv,
           W_d, b_d, W_d0, b_d0, W_d1, b_d1, W_d2, b_d2, emb, eps):
    f32 = jnp.float32
    w0p = jnp.zeros((1, 128), f32).at[0, :10].set(W_e0[:, 0])
    b0p = jnp.zeros((1, 128), f32).at[0, :10].set(b_e0)
    W1p = jnp.zeros((128, 128), f32).at[:10, :50].set(W_e1.T)
    b1p = jnp.zeros((1, 128), f32).at[0, :50].set(b_e1)
    Wmup = jnp.zeros((128, _D), f32).at[:50, :].set(W_mu.T)
    bmup = b_mu.reshape(1, _D)
    Wlvp = jnp.zeros((128, _D), f32).at[:50, :].set(W_lv.T)
    blvp = b_lv.reshape(1, _D)
    embT = emb.T
    e2 = jnp.sum(emb * emb, axis=1).reshape(1, _K)
    Wdp = jnp.zeros((_D, 128), f32).at[:, :100].set(W_d.T)
    bdp = jnp.zeros((1, 128), f32).at[0, :100].set(b_d)
    Wd0p = jnp.zeros((128, 128), f32).at[:100, :60].set(W_d0.T)
    bd0p = jnp.zeros((1, 128), f32).at[0, :60].set(b_d0)
    Wd1p = jnp.zeros((128, 128), f32).at[:60, :30].set(W_d1.T)
    bd1p = jnp.zeros((1, 128), f32).at[0, :30].set(b_d1)
    Wd2p = jnp.zeros((128, 128), f32).at[:30, :1].set(W_d2.T)
    bd2p = jnp.zeros((1, 128), f32).at[0, 0].set(b_d2[0])

    full = lambda shape: pl.BlockSpec(shape, lambda i: (0, 0))
    rows = lambda cols: pl.BlockSpec((_B, cols), lambda i: (i, 0))

    ze, zq, de, dq, cs, ss = pl.pallas_call(
        _body,
        grid=(_N // _B,),
        in_specs=[
            rows(1), rows(_D),
            full((1, 128)), full((1, 128)), full((128, 128)), full((1, 128)),
            full((128, _D)), full((1, _D)), full((128, _D)), full((1, _D)),
            full((_D, _K)), full((_K, _D)), full((1, _K)),
            full((_D, 128)), full((1, 128)), full((128, 128)), full((1, 128)),
            full((128, 128)), full((1, 128)), full((128, 128)), full((1, 128)),
        ],
        out_specs=[
            rows(_D), rows(_D), rows(1), rows(1),
            pl.BlockSpec((1, 1), lambda i: (0, 0)),
            pl.BlockSpec((1, 1), lambda i: (0, 0)),
        ],
        out_shape=[
            jax.ShapeDtypeStruct((_N, _D), f32),
            jax.ShapeDtypeStruct((_N, _D), f32),
            jax.ShapeDtypeStruct((_N, 1), f32),
            jax.ShapeDtypeStruct((_N, 1), f32),
            jax.ShapeDtypeStruct((1, 1), f32),
            jax.ShapeDtypeStruct((1, 1), f32),
        ],
    )(x, eps, w0p, b0p, W1p, b1p, Wmup, bmup, Wlvp, blvp, embT, emb, e2,
      Wdp, bdp, Wd0p, bd0p, Wd1p, bd1p, Wd2p, bd2p)

    commit_loss = 2.0 * cs[0, 0] / (_N * _D)
    som_loss = ss[0, 0] / (_N * 4 * _D)
    return ze, zq, de, dq, commit_loss, som_loss
